# baseline (speedup 1.0000x reference)
import jax
import jax.numpy as jnp
from jax.experimental import pallas as pl
from jax.experimental.pallas import tpu as pltpu


def _fused_body(adj_ref, x_ref, w_self_ref, w_agg_ref, bias_ref, out_ref, xw2_ref):
    i = pl.program_id(0)
    bm = adj_ref.shape[0]

    @pl.when(i == 0)
    def _():
        xw2_ref[...] = jnp.dot(x_ref[...], w_agg_ref[...],
                               preferred_element_type=jnp.float32)

    a = adj_ref[...]
    deg = jnp.sum(a, axis=1, keepdims=True)
    deg = jnp.where(deg == 0.0, 1.0, deg)
    y = jnp.dot(a, xw2_ref[...], preferred_element_type=jnp.float32)
    xi = x_ref[pl.ds(i * bm, bm), :]
    out = jnp.dot(xi, w_self_ref[...], preferred_element_type=jnp.float32)
    out_ref[...] = out + y / deg + bias_ref[...]


def kernel(input, adj, weight, bias):
    n, din = input.shape
    dout = weight.shape[1]
    w_self = weight[:din]
    w_agg = weight[din:]
    bias2 = bias.reshape(1, dout)
    bm = 400
    grid = (n // bm,)
    return pl.pallas_call(
        _fused_body,
        grid=grid,
        in_specs=[
            pl.BlockSpec((bm, n), lambda i: (i, 0)),
            pl.BlockSpec((n, din), lambda i: (0, 0)),
            pl.BlockSpec((din, dout), lambda i: (0, 0)),
            pl.BlockSpec((din, dout), lambda i: (0, 0)),
            pl.BlockSpec((1, dout), lambda i: (0, 0)),
        ],
        out_specs=pl.BlockSpec((bm, dout), lambda i: (i, 0)),
        out_shape=jax.ShapeDtypeStruct((n, dout), jnp.float32),
        scratch_shapes=[pltpu.VMEM((n, dout), jnp.float32)],
    )(adj, input, w_self, w_agg, bias2)
